# Initial kernel scaffold; baseline (speedup 1.0000x reference)
import jax
import jax.numpy as jnp
from jax.experimental import pallas as pl


def kernel(x, upsample):
    raise NotImplementedError("write your pallas kernel here")


# speedup vs baseline: 3.0188x; 3.0188x over previous
import jax
import jax.numpy as jnp
from jax import lax
from jax.experimental import pallas as pl
from jax.experimental.pallas import tpu as pltpu
from jax.experimental.pallas import tpu_sc as plsc

NC = 2
NS = 16
NW = NC * NS
G = 128
D = 128
B = 400000
R = B // G


def _gather_body(table_hbm, idx_hbm, out_hbm, idx_v, rows_v, sem):
    wid = lax.axis_index("s") * NC + lax.axis_index("c")

    def body(i, carry):
        row = wid + i * NW

        @pl.when(row < R)
        def _():
            pltpu.sync_copy(idx_hbm.at[row], idx_v)
            pltpu.async_copy(table_hbm.at[idx_v], rows_v, sem).wait()
            pltpu.sync_copy(rows_v, out_hbm.at[pl.ds(row * G, G)])

        return carry

    lax.fori_loop(0, (R + NW - 1) // NW, body, 0)


def kernel(x, upsample):
    idx = upsample[:, 0].astype(jnp.int32).reshape(R, G)
    table = jnp.concatenate([x, jnp.zeros((1, D), x.dtype)], axis=0)
    f = pl.kernel(
        _gather_body,
        out_type=jax.ShapeDtypeStruct((B, D), jnp.float32),
        mesh=plsc.VectorSubcoreMesh(core_axis_name="c", subcore_axis_name="s"),
        scratch_types=[
            pltpu.VMEM((G,), jnp.int32),
            pltpu.VMEM((G, D), jnp.float32),
            pltpu.SemaphoreType.DMA,
        ],
    )
    return f(table, idx)
